# baseline (device time: 23921 ns/iter reference)
import jax
import jax.numpy as jnp
from jax import lax
from jax.experimental import pallas as pl
from jax.experimental.pallas import tpu as pltpu

N_LAYERS = 3


def kernel(x, Win0, Wout0, Win1, Wout1, Win2, Wout2):
    b, dy = x.shape
    _, hx = Win0.shape

    def body(x_ref, win0_ref, wout0_ref, win1_ref, wout1_ref, win2_ref,
             wout2_ref, out_ref, h_send, h_recv, x_send, x_recv,
             send_sems, recv_sems):
        mx = lax.axis_index("x")
        my = lax.axis_index("y")
        y_partner = (mx, 1 - my)
        x_partner = (1 - mx, my)

        barrier = pltpu.get_barrier_semaphore()
        for nbr in (y_partner, x_partner):
            pl.semaphore_signal(
                barrier, inc=1, device_id=nbr,
                device_id_type=pl.DeviceIdType.MESH,
            )
        pl.semaphore_wait(barrier, 2)

        wins = [win0_ref, win1_ref, win2_ref]
        wouts = [wout0_ref, wout1_ref, wout2_ref]
        inflight = []

        def exchange(send_buf, recv_buf, slot, sem_idx, partner, partial):
            send_buf[slot] = partial
            rdma = pltpu.make_async_remote_copy(
                src_ref=send_buf.at[slot],
                dst_ref=recv_buf.at[slot],
                send_sem=send_sems.at[sem_idx],
                recv_sem=recv_sems.at[sem_idx],
                device_id=partner,
                device_id_type=pl.DeviceIdType.MESH,
            )
            rdma.start()
            inflight.append(rdma)
            rdma.wait_recv()
            return partial + recv_buf[slot]

        cur = x_ref[...].astype(jnp.bfloat16)
        for l in range(N_LAYERS):
            p = jnp.dot(
                cur,
                wins[l][...].astype(jnp.bfloat16),
                preferred_element_type=jnp.float32,
            ).astype(jnp.bfloat16)
            h = jnp.maximum(exchange(h_send, h_recv, l, 2 * l, y_partner, p), 0.0)
            q = jnp.dot(
                h,
                wouts[l][...].astype(jnp.bfloat16),
                preferred_element_type=jnp.float32,
            ).astype(jnp.bfloat16)
            cur = exchange(x_send, x_recv, l, 2 * l + 1, x_partner, q)
        out_ref[...] = cur.astype(jnp.float32)
        for rdma in inflight:
            rdma.wait_send()

    return pl.pallas_call(
        body,
        out_shape=jax.ShapeDtypeStruct((b, dy), jnp.float32),
        in_specs=[pl.BlockSpec(memory_space=pltpu.VMEM)] * 7,
        out_specs=pl.BlockSpec(memory_space=pltpu.VMEM),
        scratch_shapes=[
            pltpu.VMEM((N_LAYERS, b, hx), jnp.bfloat16),
            pltpu.VMEM((N_LAYERS, b, hx), jnp.bfloat16),
            pltpu.VMEM((N_LAYERS, b, dy), jnp.bfloat16),
            pltpu.VMEM((N_LAYERS, b, dy), jnp.bfloat16),
            pltpu.SemaphoreType.DMA((2 * N_LAYERS,)),
            pltpu.SemaphoreType.DMA((2 * N_LAYERS,)),
        ],
        compiler_params=pltpu.CompilerParams(collective_id=0),
    )(x, Win0, Wout0, Win1, Wout1, Win2, Wout2)


# device time: 22665 ns/iter; 1.0554x vs baseline; 1.0554x over previous
import jax
import jax.numpy as jnp
from jax import lax
from jax.experimental import pallas as pl
from jax.experimental.pallas import tpu as pltpu

N_LAYERS = 3
N_CHAINS = 2


def kernel(x, Win0, Wout0, Win1, Wout1, Win2, Wout2):
    b, dy = x.shape
    _, hx = Win0.shape
    half = b // N_CHAINS

    def body(x_ref, win0_ref, wout0_ref, win1_ref, wout1_ref, win2_ref,
             wout2_ref, out_ref, h_send, h_recv, x_send, x_recv,
             y_ssem, y_rsem, x_ssem, x_rsem):
        mx = lax.axis_index("x")
        my = lax.axis_index("y")
        y_partner = (mx, 1 - my)
        x_partner = (1 - mx, my)

        barrier = pltpu.get_barrier_semaphore()
        for nbr in (y_partner, x_partner):
            pl.semaphore_signal(
                barrier, inc=1, device_id=nbr,
                device_id_type=pl.DeviceIdType.MESH,
            )
        pl.semaphore_wait(barrier, 2)

        wins = [win0_ref, win1_ref, win2_ref]
        wouts = [wout0_ref, wout1_ref, wout2_ref]
        inflight = []

        def start_exchange(send_buf, recv_buf, ssem, rsem, l, c, partner, val):
            send_buf[l, c] = val
            rdma = pltpu.make_async_remote_copy(
                src_ref=send_buf.at[l, c],
                dst_ref=recv_buf.at[l, c],
                send_sem=ssem.at[l, c],
                recv_sem=rsem.at[l, c],
                device_id=partner,
                device_id_type=pl.DeviceIdType.MESH,
            )
            rdma.start()
            inflight.append(rdma)
            return rdma

        curs = [
            x_ref[c * half:(c + 1) * half, :].astype(jnp.bfloat16)
            for c in range(N_CHAINS)
        ]
        pending_x = [None] * N_CHAINS

        for l in range(N_LAYERS):
            rd_y = [None] * N_CHAINS
            pvals = [None] * N_CHAINS
            for c in range(N_CHAINS):
                if pending_x[c] is not None:
                    rd, qv, pl_ = pending_x[c]
                    rd.wait_recv()
                    curs[c] = qv + x_recv[pl_, c]
                    pending_x[c] = None
                pvals[c] = jnp.dot(
                    curs[c],
                    wins[l][...].astype(jnp.bfloat16),
                    preferred_element_type=jnp.float32,
                ).astype(jnp.bfloat16)
                rd_y[c] = start_exchange(
                    h_send, h_recv, y_ssem, y_rsem, l, c, y_partner, pvals[c]
                )
            for c in range(N_CHAINS):
                rd_y[c].wait_recv()
                hv = jnp.maximum(pvals[c] + h_recv[l, c], 0.0)
                qv = jnp.dot(
                    hv,
                    wouts[l][...].astype(jnp.bfloat16),
                    preferred_element_type=jnp.float32,
                ).astype(jnp.bfloat16)
                rd = start_exchange(
                    x_send, x_recv, x_ssem, x_rsem, l, c, x_partner, qv
                )
                pending_x[c] = (rd, qv, l)

        for c in range(N_CHAINS):
            rd, qv, pl_ = pending_x[c]
            rd.wait_recv()
            out_ref[c * half:(c + 1) * half, :] = (
                qv + x_recv[pl_, c]
            ).astype(jnp.float32)
        for rdma in inflight:
            rdma.wait_send()

    return pl.pallas_call(
        body,
        out_shape=jax.ShapeDtypeStruct((b, dy), jnp.float32),
        in_specs=[pl.BlockSpec(memory_space=pltpu.VMEM)] * 7,
        out_specs=pl.BlockSpec(memory_space=pltpu.VMEM),
        scratch_shapes=[
            pltpu.VMEM((N_LAYERS, N_CHAINS, half, hx), jnp.bfloat16),
            pltpu.VMEM((N_LAYERS, N_CHAINS, half, hx), jnp.bfloat16),
            pltpu.VMEM((N_LAYERS, N_CHAINS, half, dy), jnp.bfloat16),
            pltpu.VMEM((N_LAYERS, N_CHAINS, half, dy), jnp.bfloat16),
            pltpu.SemaphoreType.DMA((N_LAYERS, N_CHAINS)),
            pltpu.SemaphoreType.DMA((N_LAYERS, N_CHAINS)),
            pltpu.SemaphoreType.DMA((N_LAYERS, N_CHAINS)),
            pltpu.SemaphoreType.DMA((N_LAYERS, N_CHAINS)),
        ],
        compiler_params=pltpu.CompilerParams(collective_id=0),
    )(x, Win0, Wout0, Win1, Wout1, Win2, Wout2)


# device time: 22302 ns/iter; 1.0726x vs baseline; 1.0163x over previous
import jax
import jax.numpy as jnp
from jax import lax
from jax.experimental import pallas as pl
from jax.experimental.pallas import tpu as pltpu

N_LAYERS = 3
N_CHAINS = 4


def kernel(x, Win0, Wout0, Win1, Wout1, Win2, Wout2):
    b, dy = x.shape
    _, hx = Win0.shape
    half = b // N_CHAINS

    def body(x_ref, win0_ref, wout0_ref, win1_ref, wout1_ref, win2_ref,
             wout2_ref, out_ref, h_send, h_recv, x_send, x_recv,
             y_ssem, y_rsem, x_ssem, x_rsem):
        mx = lax.axis_index("x")
        my = lax.axis_index("y")
        y_partner = (mx, 1 - my)
        x_partner = (1 - mx, my)

        barrier = pltpu.get_barrier_semaphore()
        for nbr in (y_partner, x_partner):
            pl.semaphore_signal(
                barrier, inc=1, device_id=nbr,
                device_id_type=pl.DeviceIdType.MESH,
            )
        pl.semaphore_wait(barrier, 2)

        wins = [win0_ref, win1_ref, win2_ref]
        wouts = [wout0_ref, wout1_ref, wout2_ref]
        inflight = []

        def start_exchange(send_buf, recv_buf, ssem, rsem, l, c, partner, val):
            send_buf[l, c] = val
            rdma = pltpu.make_async_remote_copy(
                src_ref=send_buf.at[l, c],
                dst_ref=recv_buf.at[l, c],
                send_sem=ssem.at[l, c],
                recv_sem=rsem.at[l, c],
                device_id=partner,
                device_id_type=pl.DeviceIdType.MESH,
            )
            rdma.start()
            inflight.append(rdma)
            return rdma

        curs = [
            x_ref[c * half:(c + 1) * half, :].astype(jnp.bfloat16)
            for c in range(N_CHAINS)
        ]
        pending_x = [None] * N_CHAINS

        for l in range(N_LAYERS):
            rd_y = [None] * N_CHAINS
            pvals = [None] * N_CHAINS
            for c in range(N_CHAINS):
                if pending_x[c] is not None:
                    rd, qv, pl_ = pending_x[c]
                    rd.wait_recv()
                    curs[c] = qv + x_recv[pl_, c]
                    pending_x[c] = None
                pvals[c] = jnp.dot(
                    curs[c],
                    wins[l][...].astype(jnp.bfloat16),
                    preferred_element_type=jnp.float32,
                ).astype(jnp.bfloat16)
                rd_y[c] = start_exchange(
                    h_send, h_recv, y_ssem, y_rsem, l, c, y_partner, pvals[c]
                )
            for c in range(N_CHAINS):
                rd_y[c].wait_recv()
                hv = jnp.maximum(pvals[c] + h_recv[l, c], 0.0)
                qv = jnp.dot(
                    hv,
                    wouts[l][...].astype(jnp.bfloat16),
                    preferred_element_type=jnp.float32,
                ).astype(jnp.bfloat16)
                rd = start_exchange(
                    x_send, x_recv, x_ssem, x_rsem, l, c, x_partner, qv
                )
                pending_x[c] = (rd, qv, l)

        for c in range(N_CHAINS):
            rd, qv, pl_ = pending_x[c]
            rd.wait_recv()
            out_ref[c * half:(c + 1) * half, :] = (
                qv + x_recv[pl_, c]
            ).astype(jnp.float32)
        for rdma in inflight:
            rdma.wait_send()

    return pl.pallas_call(
        body,
        out_shape=jax.ShapeDtypeStruct((b, dy), jnp.float32),
        in_specs=[pl.BlockSpec(memory_space=pltpu.VMEM)] * 7,
        out_specs=pl.BlockSpec(memory_space=pltpu.VMEM),
        scratch_shapes=[
            pltpu.VMEM((N_LAYERS, N_CHAINS, half, hx), jnp.bfloat16),
            pltpu.VMEM((N_LAYERS, N_CHAINS, half, hx), jnp.bfloat16),
            pltpu.VMEM((N_LAYERS, N_CHAINS, half, dy), jnp.bfloat16),
            pltpu.VMEM((N_LAYERS, N_CHAINS, half, dy), jnp.bfloat16),
            pltpu.SemaphoreType.DMA((N_LAYERS, N_CHAINS)),
            pltpu.SemaphoreType.DMA((N_LAYERS, N_CHAINS)),
            pltpu.SemaphoreType.DMA((N_LAYERS, N_CHAINS)),
            pltpu.SemaphoreType.DMA((N_LAYERS, N_CHAINS)),
        ],
        compiler_params=pltpu.CompilerParams(collective_id=0),
    )(x, Win0, Wout0, Win1, Wout1, Win2, Wout2)
